# baseline (speedup 1.0000x reference)
import functools

import jax
import jax.numpy as jnp
from jax import lax
from jax.experimental import pallas as pl
from jax.experimental.pallas import tpu as pltpu
from jax.experimental.pallas import tpu_sc as plsc

D_MODEL = 64
SCALE = 8.0
G = 128
NC = 2
NS = 16
NW = NC * NS
RT = D_MODEL // 8


def _gather_scale_t(idx2d, table, n_s, n_bblk):
    n_groups = n_s * n_bblk
    gpw = n_groups // NW
    mesh = plsc.VectorSubcoreMesh(core_axis_name="c", subcore_axis_name="s")

    @functools.partial(
        pl.kernel,
        mesh=mesh,
        out_type=jax.ShapeDtypeStruct((n_s * RT * n_bblk, 8, G), jnp.float32),
        scratch_types=(
            [pltpu.VMEM((gpw, G), jnp.int32),
             pltpu.VMEM((G, D_MODEL), jnp.float32),
             pltpu.VMEM((D_MODEL, G), jnp.float32),
             pltpu.SemaphoreType.DMA]
        ),
        compiler_params=pltpu.CompilerParams(
            use_tc_tiling_on_sc=False, needs_layout_passes=False),
    )
    def k(idx_hbm, table_hbm, out_hbm, idx_v, buf, tbuf, sem):
        wid = lax.axis_index("s") * NC + lax.axis_index("c")
        g0 = wid * gpw
        pltpu.sync_copy(idx_hbm.at[pl.ds(g0, gpw)], idx_v)

        iotas = [lax.iota(jnp.int32, 16) + 16 * j for j in range(4)]

        def group_body(g, carry):
            pltpu.async_copy(table_hbm.at[idx_v.at[g]], buf, sem).wait()

            def tr_row(l, c2):
                lane = jnp.broadcast_to(l, (16,)).astype(jnp.int32)
                for j in range(D_MODEL // 16):
                    v = buf[l, pl.ds(16 * j, 16)] * SCALE
                    plsc.store_scatter(tbuf, [iotas[j], lane], v)
                return c2

            lax.fori_loop(0, G, tr_row, 0, unroll=4)

            ga = g0 + g
            base = (ga // n_bblk) * (RT * n_bblk) + (ga % n_bblk)
            for r in range(RT):
                pltpu.sync_copy(
                    tbuf.at[pl.ds(8 * r, 8)], out_hbm.at[base + r * n_bblk])
            return carry

        lax.fori_loop(0, gpw, group_body, 0)

    return k(idx2d, table)


def kernel(x, emb_weight):
    nb, n_s = x.shape
    n_bblk = nb // G
    idx2d = jnp.swapaxes(x, 0, 1).reshape(n_s * n_bblk, G).astype(jnp.int32)
    out3 = _gather_scale_t(idx2d, emb_weight, n_s, n_bblk)
    t = out3.reshape(n_s, RT, n_bblk, 8, G)
    return t.transpose(2, 4, 0, 1, 3).reshape(nb, n_s, D_MODEL)


# speedup vs baseline: 1.2024x; 1.2024x over previous
import functools

import jax
import jax.numpy as jnp
from jax import lax
from jax.experimental import pallas as pl
from jax.experimental.pallas import tpu as pltpu
from jax.experimental.pallas import tpu_sc as plsc

D_MODEL = 64
SCALE = 8.0
G = 128
NC = 2
NS = 16
NW = NC * NS
NBUF = 5
DEPTH = 2
RT = D_MODEL // 8


def _gather_scale_t(idx2d, table, n_s, n_bblk):
    n_groups = n_s * n_bblk
    gpw = n_groups // NW
    assert gpw % NBUF == 0 and gpw > NBUF
    mesh = plsc.VectorSubcoreMesh(core_axis_name="c", subcore_axis_name="s")

    @functools.partial(
        pl.kernel,
        mesh=mesh,
        out_type=jax.ShapeDtypeStruct((n_s * RT * n_bblk, 8, G), jnp.float32),
        scratch_types=(
            [pltpu.VMEM((gpw, G), jnp.int32)]
            + [pltpu.VMEM((G, D_MODEL), jnp.float32) for _ in range(NBUF)]
            + [pltpu.VMEM((D_MODEL, G), jnp.float32) for _ in range(NBUF)]
            + [pltpu.SemaphoreType.DMA for _ in range(2 * NBUF)]
        ),
        compiler_params=pltpu.CompilerParams(
            use_tc_tiling_on_sc=False, needs_layout_passes=False),
    )
    def k(idx_hbm, table_hbm, out_hbm, idx_v, *rest):
        bufs = rest[:NBUF]
        tbufs = rest[NBUF:2 * NBUF]
        gsem = rest[2 * NBUF:3 * NBUF]
        ssem = rest[3 * NBUF:]
        wid = lax.axis_index("s") * NC + lax.axis_index("c")
        g0 = wid * gpw
        pltpu.sync_copy(idx_hbm.at[pl.ds(g0, gpw)], idx_v)

        def gather(g, b):
            pltpu.async_copy(table_hbm.at[idx_v.at[g]], bufs[b], gsem[b])

        def wait_gather(g, b):
            pltpu.make_async_copy(
                table_hbm.at[idx_v.at[g]], bufs[b], gsem[b]).wait()

        def out_tile(g, r):
            ga = g0 + g
            return (ga // n_bblk) * (RT * n_bblk) + r * n_bblk + (ga % n_bblk)

        def store(g, b):
            for r in range(RT):
                pltpu.async_copy(
                    tbufs[b].at[pl.ds(8 * r, 8)], out_hbm.at[out_tile(g, r)],
                    ssem[b])

        def wait_store(g, b):
            for r in range(RT):
                pltpu.make_async_copy(
                    tbufs[b].at[pl.ds(8 * r, 8)], out_hbm.at[out_tile(g, r)],
                    ssem[b]).wait()

        iotas = [lax.iota(jnp.int32, 16) + 16 * j for j in range(4)]

        def block(kb, carry):
            for b in range(NBUF):
                g = kb * NBUF + b
                wait_gather(g, b)

                def tr_row(l, c2):
                    lane = jnp.broadcast_to(l, (16,)).astype(jnp.int32)
                    for j in range(D_MODEL // 16):
                        v = bufs[b][l, pl.ds(16 * j, 16)] * SCALE
                        plsc.store_scatter(tbufs[b], [iotas[j], lane], v)
                    return c2

                lax.fori_loop(0, G, tr_row, 0, unroll=4)
                store(g, b)

                gn = g + DEPTH
                bn = (b + DEPTH) % NBUF

                @pl.when(jnp.logical_and(gn >= NBUF, gn < gpw))
                def _():
                    wait_store(gn - NBUF, bn)

                @pl.when(gn < gpw)
                def _():
                    gather(gn, bn)
            return carry

        for b in range(DEPTH):
            gather(b, b)
        lax.fori_loop(0, gpw // NBUF, block, 0)
        for b in range(NBUF):
            wait_store(gpw - NBUF + b, b)

    return k(idx2d, table)


def kernel(x, emb_weight):
    nb, n_s = x.shape
    n_bblk = nb // G
    idx2d = jnp.swapaxes(x, 0, 1).reshape(n_s * n_bblk, G).astype(jnp.int32)
    out3 = _gather_scale_t(idx2d, emb_weight, n_s, n_bblk)
    t = out3.reshape(n_s, RT, n_bblk, 8, G)
    return t.transpose(2, 4, 0, 1, 3).reshape(nb, n_s, D_MODEL)


# speedup vs baseline: 1.8619x; 1.5485x over previous
import functools

import jax
import jax.numpy as jnp
from jax import lax
from jax.experimental import pallas as pl
from jax.experimental.pallas import tpu as pltpu
from jax.experimental.pallas import tpu_sc as plsc

D_MODEL = 64
SCALE = 8.0
G = 128
NC = 2
NS = 16
NW = NC * NS
NBUF = 5
DEPTH = 2
RT = D_MODEL // 8


def _gather_scale_t(idx2d, table, n_s, n_bblk):
    n_groups = n_s * n_bblk
    gpw = n_groups // NW
    assert gpw % NBUF == 0 and gpw > NBUF
    mesh = plsc.VectorSubcoreMesh(core_axis_name="c", subcore_axis_name="s")

    @functools.partial(
        pl.kernel,
        mesh=mesh,
        out_type=jax.ShapeDtypeStruct((n_s * RT * n_bblk, 8, G), jnp.float32),
        scratch_types=(
            [pltpu.VMEM((gpw, G), jnp.int32)]
            + [pltpu.VMEM((G, D_MODEL), jnp.float32) for _ in range(NBUF)]
            + [pltpu.VMEM((D_MODEL, G + 1), jnp.float32) for _ in range(NBUF)]
            + [pltpu.SemaphoreType.DMA for _ in range(2 * NBUF)]
        ),
        compiler_params=pltpu.CompilerParams(
            use_tc_tiling_on_sc=False, needs_layout_passes=False),
    )
    def k(idx_hbm, table_hbm, out_hbm, idx_v, *rest):
        bufs = rest[:NBUF]
        tbufs = rest[NBUF:2 * NBUF]
        gsem = rest[2 * NBUF:3 * NBUF]
        ssem = rest[3 * NBUF:]
        wid = lax.axis_index("s") * NC + lax.axis_index("c")
        g0 = wid * gpw
        pltpu.sync_copy(idx_hbm.at[pl.ds(g0, gpw)], idx_v)

        def gather(g, b):
            pltpu.async_copy(table_hbm.at[idx_v.at[g]], bufs[b], gsem[b])

        def wait_gather(g, b):
            pltpu.make_async_copy(
                table_hbm.at[idx_v.at[g]], bufs[b], gsem[b]).wait()

        def out_tile(g, r):
            ga = g0 + g
            u = ga % 8
            cell = ga // 8
            c = cell % n_bblk
            sr = cell // n_bblk
            s = sr * 8 + u
            return s * (RT * n_bblk) + r * n_bblk + c

        def store(g, b):
            for r in range(RT):
                pltpu.async_copy(
                    tbufs[b].at[pl.ds(8 * r, 8), pl.ds(0, G)],
                    out_hbm.at[out_tile(g, r)], ssem[b])

        def wait_store(g, b):
            for r in range(RT):
                pltpu.make_async_copy(
                    tbufs[b].at[pl.ds(8 * r, 8), pl.ds(0, G)],
                    out_hbm.at[out_tile(g, r)], ssem[b]).wait()

        iotas = [lax.iota(jnp.int32, 16) + 16 * j for j in range(4)]

        def block(kb, carry):
            for b in range(NBUF):
                g = kb * NBUF + b
                wait_gather(g, b)

                def tr_row(l, c2):
                    lane = jnp.broadcast_to(l, (16,)).astype(jnp.int32)
                    for j in range(D_MODEL // 16):
                        v = bufs[b][l, pl.ds(16 * j, 16)] * SCALE
                        plsc.store_scatter(tbufs[b], [iotas[j], lane], v)
                    return c2

                lax.fori_loop(0, G, tr_row, 0, unroll=4)
                store(g, b)

                gn = g + DEPTH
                bn = (b + DEPTH) % NBUF

                @pl.when(jnp.logical_and(gn >= NBUF, gn < gpw))
                def _():
                    wait_store(gn - NBUF, bn)

                @pl.when(gn < gpw)
                def _():
                    gather(gn, bn)
            return carry

        for b in range(DEPTH):
            gather(b, b)
        lax.fori_loop(0, gpw // NBUF, block, 0)
        for b in range(NBUF):
            wait_store(gpw - NBUF + b, b)

    return k(idx2d, table)


def kernel(x, emb_weight):
    nb, n_s = x.shape
    n_bblk = nb // G
    assert n_s % 8 == 0
    idx2d = (jnp.swapaxes(x, 0, 1)
             .reshape(n_s // 8, 8, n_bblk, G)
             .transpose(0, 2, 1, 3)
             .reshape(n_s * n_bblk, G)
             .astype(jnp.int32))
    out3 = _gather_scale_t(idx2d, emb_weight, n_s, n_bblk)
    t = out3.reshape(n_s, RT, n_bblk, 8, G)
    return t.transpose(2, 4, 0, 1, 3).reshape(nb, n_s, D_MODEL)


# speedup vs baseline: 2.2615x; 1.2146x over previous
import functools

import jax
import jax.numpy as jnp
from jax import lax
from jax.experimental import pallas as pl
from jax.experimental.pallas import tpu as pltpu
from jax.experimental.pallas import tpu_sc as plsc

D_MODEL = 64
SCALE = 8.0
G = 128
NC = 2
NS = 16
NW = NC * NS
NBUF = 5
DEPTH = 2
RT = D_MODEL // 8


def _gather_scale_t(idx2d, table, n_s, n_bblk):
    n_groups = n_s * n_bblk
    gpw = n_groups // NW
    assert gpw % NBUF == 0 and gpw > NBUF
    mesh = plsc.VectorSubcoreMesh(core_axis_name="c", subcore_axis_name="s")

    @functools.partial(
        pl.kernel,
        mesh=mesh,
        out_type=jax.ShapeDtypeStruct((n_s * RT * n_bblk, 8, G), jnp.float32),
        scratch_types=(
            [pltpu.VMEM((gpw, G), jnp.int32)]
            + [pltpu.VMEM((G, D_MODEL), jnp.float32) for _ in range(NBUF)]
            + [pltpu.VMEM((D_MODEL, G + 1), jnp.float32) for _ in range(NBUF)]
            + [pltpu.SemaphoreType.DMA for _ in range(2 * NBUF)]
        ),
        compiler_params=pltpu.CompilerParams(
            use_tc_tiling_on_sc=False, needs_layout_passes=False),
    )
    def k(idx_hbm, table_hbm, out_hbm, idx_v, *rest):
        bufs = rest[:NBUF]
        tbufs = rest[NBUF:2 * NBUF]
        gsem = rest[2 * NBUF:3 * NBUF]
        ssem = rest[3 * NBUF:]
        wid = lax.axis_index("s") * NC + lax.axis_index("c")
        g0 = wid * gpw
        pltpu.sync_copy(idx_hbm.at[pl.ds(g0, gpw)], idx_v)

        def gather(g, b):
            pltpu.async_copy(table_hbm.at[idx_v.at[g]], bufs[b], gsem[b])

        def wait_gather(g, b):
            pltpu.make_async_copy(
                table_hbm.at[idx_v.at[g]], bufs[b], gsem[b]).wait()

        def out_tile(g, r):
            ga = g0 + g
            u = ga % 8
            cell = ga // 8
            c = cell % n_bblk
            sr = cell // n_bblk
            s = sr * 8 + u
            return s * (RT * n_bblk) + r * n_bblk + c

        def store(g, b):
            for r in range(RT):
                pltpu.async_copy(
                    tbufs[b].at[pl.ds(8 * r, 8), pl.ds(0, G)],
                    out_hbm.at[out_tile(g, r)], ssem[b])

        def wait_store(g, b):
            for r in range(RT):
                pltpu.make_async_copy(
                    tbufs[b].at[pl.ds(8 * r, 8), pl.ds(0, G)],
                    out_hbm.at[out_tile(g, r)], ssem[b]).wait()

        iotas = [lax.iota(jnp.int32, 16) + 16 * j for j in range(4)]

        def block(kb, carry):
            for b in range(NBUF):
                g = kb * NBUF + b
                wait_gather(g, b)

                def tr_row(l, c2):
                    lane = jnp.broadcast_to(l, (16,)).astype(jnp.int32)
                    vals = [bufs[b][l, pl.ds(16 * j, 16)]
                            for j in range(D_MODEL // 16)]
                    vals = [v * SCALE for v in vals]
                    for j in range(D_MODEL // 16):
                        plsc.store_scatter(tbufs[b], [iotas[j], lane], vals[j])
                    return c2

                lax.fori_loop(0, G, tr_row, 0, unroll=4)
                store(g, b)

                gn = g + DEPTH
                bn = (b + DEPTH) % NBUF

                @pl.when(jnp.logical_and(gn >= NBUF, gn < gpw))
                def _():
                    wait_store(gn - NBUF, bn)

                @pl.when(gn < gpw)
                def _():
                    gather(gn, bn)
            return carry

        for b in range(DEPTH):
            gather(b, b)
        lax.fori_loop(0, gpw // NBUF, block, 0)
        for b in range(NBUF):
            wait_store(gpw - NBUF + b, b)

    return k(idx2d, table)


def kernel(x, emb_weight):
    nb, n_s = x.shape
    n_bblk = nb // G
    assert n_s % 8 == 0
    idx2d = (jnp.swapaxes(x, 0, 1)
             .reshape(n_s // 8, 8, n_bblk, G)
             .transpose(0, 2, 1, 3)
             .reshape(n_s * n_bblk, G)
             .astype(jnp.int32))
    out3 = _gather_scale_t(idx2d, emb_weight, n_s, n_bblk)
    t = out3.reshape(n_s, RT, n_bblk, 8, G)
    return t.transpose(2, 4, 0, 1, 3).reshape(nb, n_s, D_MODEL)


# speedup vs baseline: 2.2660x; 1.0020x over previous
import functools

import jax
import jax.numpy as jnp
from jax import lax
from jax.experimental import pallas as pl
from jax.experimental.pallas import tpu as pltpu
from jax.experimental.pallas import tpu_sc as plsc

D_MODEL = 64
SCALE = 8.0
G = 128
NC = 2
NS = 16
NW = NC * NS
NBUF = 5
DEPTH = 2
RT = D_MODEL // 8


def _gather_scale_t(idx2d, table, n_s, n_bblk):
    n_groups = n_s * n_bblk
    gpw = n_groups // NW
    assert gpw % NBUF == 0 and gpw > NBUF
    mesh = plsc.VectorSubcoreMesh(core_axis_name="c", subcore_axis_name="s")

    @functools.partial(
        pl.kernel,
        mesh=mesh,
        out_type=jax.ShapeDtypeStruct((n_s * RT * n_bblk, 8, G), jnp.float32),
        scratch_types=(
            [pltpu.VMEM((gpw, G), jnp.int32)]
            + [pltpu.VMEM((G, D_MODEL), jnp.float32) for _ in range(NBUF)]
            + [pltpu.VMEM((D_MODEL, G + 1), jnp.float32) for _ in range(NBUF)]
            + [pltpu.SemaphoreType.DMA for _ in range(2 * NBUF)]
        ),
        compiler_params=pltpu.CompilerParams(
            use_tc_tiling_on_sc=False, needs_layout_passes=False),
    )
    def k(idx_hbm, table_hbm, out_hbm, idx_v, *rest):
        bufs = rest[:NBUF]
        tbufs = rest[NBUF:2 * NBUF]
        gsem = rest[2 * NBUF:3 * NBUF]
        ssem = rest[3 * NBUF:]
        wid = lax.axis_index("s") * NC + lax.axis_index("c")
        g0 = wid * gpw
        pltpu.sync_copy(idx_hbm.at[pl.ds(g0, gpw)], idx_v)

        def gather(g, b):
            pltpu.async_copy(table_hbm.at[idx_v.at[g]], bufs[b], gsem[b])

        def wait_gather(g, b):
            pltpu.make_async_copy(
                table_hbm.at[idx_v.at[g]], bufs[b], gsem[b]).wait()

        def out_tile(g, r):
            ga = g0 + g
            u = ga % 8
            cell = ga // 8
            c = cell % n_bblk
            sr = cell // n_bblk
            s = sr * 8 + u
            return s * (RT * n_bblk) + r * n_bblk + c

        def store(g, b):
            for r in range(RT):
                pltpu.async_copy(
                    tbufs[b].at[pl.ds(8 * r, 8), pl.ds(0, G)],
                    out_hbm.at[out_tile(g, r)], ssem[b])

        def wait_store(g, b):
            for r in range(RT):
                pltpu.make_async_copy(
                    tbufs[b].at[pl.ds(8 * r, 8), pl.ds(0, G)],
                    out_hbm.at[out_tile(g, r)], ssem[b]).wait()

        iotas = [lax.iota(jnp.int32, 16) + 16 * j for j in range(4)]

        def block(kb, carry):
            for b in range(NBUF):
                g = kb * NBUF + b
                wait_gather(g, b)

                gn = g + DEPTH
                bn = (b + DEPTH) % NBUF

                @pl.when(jnp.logical_and(gn >= NBUF, gn < gpw))
                def _():
                    wait_store(gn - NBUF, bn)

                @pl.when(gn < gpw)
                def _():
                    gather(gn, bn)

                def tr_row(l, c2):
                    lane = jnp.broadcast_to(l, (16,)).astype(jnp.int32)
                    vals = [bufs[b][l, pl.ds(16 * j, 16)]
                            for j in range(D_MODEL // 16)]
                    vals = [v * SCALE for v in vals]
                    for j in range(D_MODEL // 16):
                        plsc.store_scatter(tbufs[b], [iotas[j], lane], vals[j])
                    return c2

                lax.fori_loop(0, G, tr_row, 0, unroll=4)
                store(g, b)
            return carry

        for b in range(DEPTH):
            gather(b, b)
        lax.fori_loop(0, gpw // NBUF, block, 0)
        for b in range(NBUF):
            wait_store(gpw - NBUF + b, b)

    return k(idx2d, table)


def kernel(x, emb_weight):
    nb, n_s = x.shape
    n_bblk = nb // G
    assert n_s % 8 == 0
    idx2d = (jnp.swapaxes(x, 0, 1)
             .reshape(n_s // 8, 8, n_bblk, G)
             .transpose(0, 2, 1, 3)
             .reshape(n_s * n_bblk, G)
             .astype(jnp.int32))
    out3 = _gather_scale_t(idx2d, emb_weight, n_s, n_bblk)
    t = out3.reshape(n_s, RT, n_bblk, 8, G)
    return t.transpose(2, 4, 0, 1, 3).reshape(nb, n_s, D_MODEL)


# speedup vs baseline: 2.3778x; 1.0493x over previous
import functools

import jax
import jax.numpy as jnp
from jax import lax
from jax.experimental import pallas as pl
from jax.experimental.pallas import tpu as pltpu
from jax.experimental.pallas import tpu_sc as plsc

D_MODEL = 64
SCALE = 8.0
G = 128
NC = 2
NS = 16
NW = NC * NS
NBUF = 5
DEPTH = 2
RT = D_MODEL // 8


def _gather_scale_t(idx2d, table, n_s, n_bblk):
    n_groups = n_s * n_bblk
    gpw = n_groups // NW
    assert gpw % NBUF == 0 and gpw > NBUF
    mesh = plsc.VectorSubcoreMesh(core_axis_name="c", subcore_axis_name="s")

    @functools.partial(
        pl.kernel,
        mesh=mesh,
        out_type=jax.ShapeDtypeStruct((n_s * RT * n_bblk, 8, G), jnp.float32),
        scratch_types=(
            [pltpu.VMEM((gpw, G), jnp.int32)]
            + [pltpu.VMEM((G, D_MODEL), jnp.float32) for _ in range(NBUF)]
            + [pltpu.VMEM((D_MODEL, G + 1), jnp.float32) for _ in range(NBUF)]
            + [pltpu.SemaphoreType.DMA for _ in range(2 * NBUF)]
        ),
        compiler_params=pltpu.CompilerParams(
            use_tc_tiling_on_sc=False, needs_layout_passes=False),
    )
    def k(idx_hbm, table_hbm, out_hbm, idx_v, *rest):
        bufs = rest[:NBUF]
        tbufs = rest[NBUF:2 * NBUF]
        gsem = rest[2 * NBUF:3 * NBUF]
        ssem = rest[3 * NBUF:]
        wid = lax.axis_index("s") * NC + lax.axis_index("c")
        g0 = wid * gpw
        pltpu.sync_copy(idx_hbm.at[pl.ds(g0, gpw)], idx_v)

        def gather(g, b):
            pltpu.async_copy(table_hbm.at[idx_v.at[g]], bufs[b], gsem[b])

        def wait_gather(g, b):
            pltpu.make_async_copy(
                table_hbm.at[idx_v.at[g]], bufs[b], gsem[b]).wait()

        def out_tile(g, r):
            ga = g0 + g
            u = ga % 8
            cell = ga // 8
            c = cell % n_bblk
            sr = cell // n_bblk
            s = sr * 8 + u
            return s * (RT * n_bblk) + r * n_bblk + c

        def store(g, b):
            for r in range(RT):
                pltpu.async_copy(
                    tbufs[b].at[pl.ds(8 * r, 8), pl.ds(0, G)],
                    out_hbm.at[out_tile(g, r)], ssem[b])

        def wait_store(g, b):
            for r in range(RT):
                pltpu.make_async_copy(
                    tbufs[b].at[pl.ds(8 * r, 8), pl.ds(0, G)],
                    out_hbm.at[out_tile(g, r)], ssem[b]).wait()

        iotas = [lax.iota(jnp.int32, 16) + 16 * j for j in range(4)]

        def block(kb, carry):
            for b in range(NBUF):
                g = kb * NBUF + b
                wait_gather(g, b)

                gn = g + DEPTH
                bn = (b + DEPTH) % NBUF

                @pl.when(jnp.logical_and(gn >= NBUF, gn < gpw))
                def _():
                    wait_store(gn - NBUF, bn)

                @pl.when(gn < gpw)
                def _():
                    gather(gn, bn)

                def tr_row(l2, c2):
                    la = 2 * l2
                    lanes = [jnp.broadcast_to(la + i, (16,)).astype(jnp.int32)
                             for i in range(2)]
                    vals = [bufs[b][la + i, pl.ds(16 * j, 16)]
                            for i in range(2)
                            for j in range(D_MODEL // 16)]
                    vals = [v * SCALE for v in vals]
                    for i in range(2):
                        for j in range(D_MODEL // 16):
                            plsc.store_scatter(
                                tbufs[b], [iotas[j], lanes[i]],
                                vals[i * (D_MODEL // 16) + j])
                    return c2

                lax.fori_loop(0, G // 2, tr_row, 0, unroll=2)
                store(g, b)
            return carry

        for b in range(DEPTH):
            gather(b, b)
        lax.fori_loop(0, gpw // NBUF, block, 0)
        for b in range(NBUF):
            wait_store(gpw - NBUF + b, b)

    return k(idx2d, table)


def kernel(x, emb_weight):
    nb, n_s = x.shape
    n_bblk = nb // G
    assert n_s % 8 == 0
    idx2d = (jnp.swapaxes(x, 0, 1)
             .reshape(n_s // 8, 8, n_bblk, G)
             .transpose(0, 2, 1, 3)
             .reshape(n_s * n_bblk, G)
             .astype(jnp.int32))
    out3 = _gather_scale_t(idx2d, emb_weight, n_s, n_bblk)
    t = out3.reshape(n_s, RT, n_bblk, 8, G)
    return t.transpose(2, 4, 0, 1, 3).reshape(nb, n_s, D_MODEL)
